# baseline (speedup 1.0000x reference)
import functools

import jax
import jax.numpy as jnp
from jax import lax
from jax.experimental import pallas as pl
from jax.experimental.pallas import tpu as pltpu
from jax.experimental.pallas import tpu_sc as plsc

_L = 16


def kernel(x, row_table, col_table):
    bs, _, h, w = x.shape
    out_n = row_table.shape[1]
    c_total = 2 * out_n
    mesh = plsc.VectorSubcoreMesh(core_axis_name="c", subcore_axis_name="s")

    @functools.partial(
        pl.kernel,
        mesh=mesh,
        out_type=jax.ShapeDtypeStruct((bs, h, w, c_total), jnp.float32),
        scratch_types=[
            pltpu.VMEM((h, out_n), jnp.float32),
            pltpu.VMEM((out_n,), jnp.float32),
            pltpu.VMEM((w, c_total), jnp.float32),
            pltpu.SemaphoreType.DMA,
        ],
    )
    def sck(row_hbm, col_hbm, out_hbm, rows_v, colrow_v, slice_v, sem):
        wid = lax.axis_index("s") * 2 + lax.axis_index("c")
        pltpu.sync_copy(row_hbm.at[pl.ds(0, h)], rows_v)
        pltpu.sync_copy(col_hbm.at[wid], colrow_v)
        for q in range(w):
            for j in range(out_n // _L):
                slice_v[q, pl.ds(j * _L, _L)] = rows_v[q, pl.ds(j * _L, _L)]
        for j in range(out_n // _L):
            v = colrow_v[pl.ds(j * _L, _L)]
            for q in range(w):
                slice_v[q, pl.ds(out_n + j * _L, _L)] = v
        copies = [
            pltpu.make_async_copy(slice_v, out_hbm.at[b, wid], sem)
            for b in range(bs)
        ]
        for c in copies:
            c.start()
        for c in copies:
            c.wait()

    y = sck(row_table, col_table)
    return jnp.transpose(y, (0, 3, 1, 2))


# speedup vs baseline: 2.0565x; 2.0565x over previous
import jax
import jax.numpy as jnp
from jax.experimental import pallas as pl


def _body(row_ref, col_ref, out_ref):
    h, out_n = row_ref.shape
    bblk = out_ref.shape[0]
    top = jnp.broadcast_to(row_ref[...][None, None, :, :], (bblk, h, h, out_n))
    bot = jnp.broadcast_to(col_ref[...][None, :, None, :], (bblk, h, h, out_n))
    out_ref[:, :, :, :out_n] = top
    out_ref[:, :, :, out_n:] = bot


def kernel(x, row_table, col_table):
    bs, _, h, w = x.shape
    out_n = row_table.shape[1]
    c_total = 2 * out_n
    bblk = 2

    y = pl.pallas_call(
        _body,
        grid=(bs // bblk,),
        in_specs=[
            pl.BlockSpec((h, out_n), lambda b: (0, 0)),
            pl.BlockSpec((w, out_n), lambda b: (0, 0)),
        ],
        out_specs=pl.BlockSpec((bblk, h, w, c_total), lambda b: (b, 0, 0, 0)),
        out_shape=jax.ShapeDtypeStruct((bs, h, w, c_total), jnp.float32),
    )(row_table, col_table)
    return jnp.transpose(y, (0, 3, 1, 2))
